# baseline (speedup 1.0000x reference)
import functools

import jax
import jax.numpy as jnp
from jax.experimental import pallas as pl
from jax.experimental.pallas import tpu as pltpu
from jax.experimental.pallas import tpu_sc as plsc

F32 = jnp.float32

_SC_NC = 2
_SC_NS = 16
_SC_NW = _SC_NC * _SC_NS
_SC_L = 128


def _wpad(D):
    return ((2 * D + 3 + 127) // 128) * 128


def _full_spec(shape):
    return pl.BlockSpec(shape, lambda *a, n=len(shape): (0,) * n)


def _bspec(shape):
    return pl.BlockSpec((1,) + tuple(shape[1:]),
                        lambda *a, n=len(shape): (a[0],) + (0,) * (n - 1))


def _ptb_wargs(p):
    ptl = p['ptl']
    dps = [p['lin1'], ptl['q'], ptl['k'], ptl['v'],
           ptl['pos']['l1'], ptl['pos']['l2'],
           ptl['attn']['l1'], ptl['attn']['l2'], p['lin2']]
    return [x for dp in dps for x in (dp['W'], dp['b'].reshape(1, -1))]


def _sc_gather(table, idx_flat, C):
    M = idx_flat.shape[0]
    nck = M // (_SC_NW * _SC_L)
    idx3 = idx_flat.reshape(_SC_NW, nck, _SC_L)
    mesh = plsc.VectorSubcoreMesh(core_axis_name="c", subcore_axis_name="s")

    def body(idx_hbm, tab_hbm, out_hbm, idx_v, rows_v, sem):
        wid = jax.lax.axis_index("s") * _SC_NC + jax.lax.axis_index("c")

        def chunk(c, carry):
            pltpu.sync_copy(idx_hbm.at[wid, c], idx_v)
            pltpu.async_copy(tab_hbm.at[idx_v], rows_v, sem).wait()
            pltpu.sync_copy(
                rows_v, out_hbm.at[pl.ds((wid * nck + c) * _SC_L, _SC_L)])
            return carry

        jax.lax.fori_loop(0, nck, chunk, 0)

    fn = pl.kernel(
        body,
        mesh=mesh,
        out_type=jax.ShapeDtypeStruct((M, C), F32),
        scratch_types=[
            pltpu.VMEM((_SC_L,), jnp.int32),
            pltpu.VMEM((_SC_L, C), F32),
            pltpu.SemaphoreType.DMA,
        ],
    )
    return fn(idx3, table)


def _gather_rows(table2d, idxg, C):
    B, K, Nq = idxg.shape
    M = B * K * Nq
    Mp = ((M + 4095) // 4096) * 4096
    flat = idxg.reshape(M)
    if Mp != M:
        flat = jnp.pad(flat, (0, Mp - M))
    g = _sc_gather(table2d, flat, C)
    return g[:M].reshape(B, K, Nq, C)


_KNN_CFGS = (
    (2048, 2048, 16, False, False, True),
    (512, 512, 16, False, False, True),
    (128, 128, 8, False, False, True),
    (32, 32, 4, False, True, False),
    (8, 8, 2, False, True, False),
    (512, 2048, 16, False, True, False),
    (128, 512, 8, False, True, False),
    (32, 128, 4, False, True, False),
    (8, 32, 2, False, True, False),
    (32, 8, 3, True, True, False),
    (128, 32, 3, True, True, False),
    (512, 128, 3, True, True, False),
    (2048, 512, 3, True, True, False),
)


def _knn_all_kernel(xyz_ref, xyzT_ref, *out_refs):
    b = pl.program_id(0)
    oi = 0
    for (Nq, Nk, K, wd, wl, wg) in _KNN_CFGS:
        dref = iref = gref = None
        if wd:
            dref = out_refs[oi]
            oi += 1
        if wl:
            iref = out_refs[oi]
            oi += 1
        if wg:
            gref = out_refs[oi]
            oi += 1
        R = min(Nq, 512)
        for blk in range(Nq // R):
            r0 = blk * R
            qx = xyz_ref[0, r0:r0 + R, 0][:, None]
            qy = xyz_ref[0, r0:r0 + R, 1][:, None]
            qz = xyz_ref[0, r0:r0 + R, 2][:, None]
            kx = xyzT_ref[0, 0, :Nk][None, :]
            ky = xyzT_ref[0, 1, :Nk][None, :]
            kz = xyzT_ref[0, 2, :Nk][None, :]
            dxv = qx - kx
            dyv = qy - ky
            dzv = qz - kz
            cur = dxv * dxv + dyv * dyv + dzv * dzv
            iota = jax.lax.broadcasted_iota(jnp.int32, (R, Nk), 1)
            for kk in range(K):
                m = jnp.min(cur, axis=1)
                am = jnp.min(jnp.where(cur == m[:, None], iota, Nk), axis=1)
                if wd:
                    dref[0, kk, r0:r0 + R] = m
                if wl:
                    iref[0, kk, r0:r0 + R] = am
                if wg:
                    gref[0, kk, r0:r0 + R] = am + b * Nk
                if kk < K - 1:
                    cur = jnp.where(iota == am[:, None],
                                    jnp.float32(jnp.inf), cur)


def _knn_all(xyz0):
    B = xyz0.shape[0]
    xyzT = jnp.transpose(xyz0, (0, 2, 1))
    out_shape = []
    out_specs = []
    for (Nq, Nk, K, wd, wl, wg) in _KNN_CFGS:
        if wd:
            out_shape.append(jax.ShapeDtypeStruct((B, K, Nq), F32))
            out_specs.append(_bspec((B, K, Nq)))
        if wl:
            out_shape.append(jax.ShapeDtypeStruct((B, K, Nq), jnp.int32))
            out_specs.append(_bspec((B, K, Nq)))
        if wg:
            out_shape.append(jax.ShapeDtypeStruct((B, K, Nq), jnp.int32))
            out_specs.append(_bspec((B, K, Nq)))
    return pl.pallas_call(
        _knn_all_kernel,
        grid=(B,),
        in_specs=[_bspec(xyz0.shape), _bspec(xyzT.shape)],
        out_specs=out_specs,
        out_shape=out_shape,
    )(xyz0, xyzT)


def _attn_tail(qb, xq, pairs, wp1, bp1, wp2, bp2, wa1, ba1, wa2, ba2):
    a_list = []
    vp_list = []
    for kg, vg, nx in pairs:
        pd = xq - nx
        h = jnp.maximum(jnp.dot(pd, wp1[...], preferred_element_type=F32)
                        + bp1[...], 0.0)
        pos = jnp.dot(h, wp2[...], preferred_element_type=F32) + bp2[...]
        ain = qb - kg + pos
        h2 = jnp.maximum(jnp.dot(ain, wa1[...], preferred_element_type=F32)
                         + ba1[...], 0.0)
        a = jnp.dot(h2, wa2[...], preferred_element_type=F32) + ba2[...]
        a_list.append(a)
        vp_list.append(vg + pos)
    m = a_list[0]
    for a in a_list[1:]:
        m = jnp.maximum(m, a)
    es = [jnp.exp(a - m) for a in a_list]
    s = es[0]
    for e in es[1:]:
        s = s + e
    num = es[0] * vp_list[0]
    for kk in range(1, len(es)):
        num = num + es[kk] * vp_list[kk]
    return num / s


def _qkv_from(y_in, xyz, wrefs, kv_ref, *, D):
    (w1, b1, wq, bq, wk, bk, wv, bv) = wrefs
    y = jnp.dot(y_in, w1[...], preferred_element_type=F32) + b1[...]
    q = jnp.dot(y, wq[...], preferred_element_type=F32) + bq[...]
    kv_ref[0, :, :D] = jnp.dot(y, wk[...], preferred_element_type=F32) + bk[...]
    kv_ref[0, :, D:2 * D] = (
        jnp.dot(y, wv[...], preferred_element_type=F32) + bv[...])
    kv_ref[0, :, 2 * D:2 * D + 3] = xyz
    return q


def _ptb_from_refs(f, xyz, idx_ref, wrefs, K):
    (w1, b1, wq, bq, wk, bk, wv, bv,
     wp1, bp1, wp2, bp2, wa1, ba1, wa2, ba2, w2, b2) = wrefs
    N = f.shape[0]
    y = jnp.dot(f, w1[...], preferred_element_type=F32) + b1[...]
    q = jnp.dot(y, wq[...], preferred_element_type=F32) + bq[...]
    k = jnp.dot(y, wk[...], preferred_element_type=F32) + bk[...]
    v = jnp.dot(y, wv[...], preferred_element_type=F32) + bv[...]
    iota = jax.lax.broadcasted_iota(jnp.int32, (N, N), 1)
    pairs = []
    for kk in range(K):
        oh = (idx_ref[0, kk, :][:, None] == iota).astype(F32)
        kg = jnp.dot(oh, k, preferred_element_type=F32)
        vg = jnp.dot(oh, v, preferred_element_type=F32)
        nx = jnp.dot(oh, xyz, preferred_element_type=F32)
        pairs.append((kg, vg, nx))
    attn = _attn_tail(q, xyz, pairs, wp1, bp1, wp2, bp2, wa1, ba1, wa2, ba2)
    return f + jnp.dot(attn, w2[...], preferred_element_type=F32) + b2[...]


def _td_from(fprev, itd_ref, wtd, btd, Ktd, Nn):
    Nprev = fprev.shape[0]
    iota = jax.lax.broadcasted_iota(jnp.int32, (Nn, Nprev), 1)
    acc = None
    for kk in range(Ktd):
        oh = (itd_ref[0, kk, :][:, None] == iota).astype(F32)
        g = jnp.dot(oh, fprev, preferred_element_type=F32)
        h = jnp.maximum(
            jnp.dot(g, wtd[...], preferred_element_type=F32) + btd[...], 0.0)
        acc = h if acc is None else jnp.maximum(acc, h)
    return acc


def _tu_from(fc_in, ff_in, d_ref, itu_ref, wl1, bl1, wl2, bl2):
    Nc = fc_in.shape[0]
    Nf = ff_in.shape[0]
    fc = jnp.dot(fc_in, wl1[...], preferred_element_type=F32) + bl1[...]
    ff = jnp.dot(ff_in, wl2[...], preferred_element_type=F32) + bl2[...]
    ws = [1.0 / (d_ref[0, kk, :] + 1e-8) for kk in range(3)]
    s = ws[0] + ws[1] + ws[2]
    iota = jax.lax.broadcasted_iota(jnp.int32, (Nf, Nc), 1)
    acc = None
    for kk in range(3):
        oh = (itu_ref[0, kk, :][:, None] == iota).astype(F32)
        g = jnp.dot(oh, fc, preferred_element_type=F32)
        t = (ws[kk] / s)[:, None] * g
        acc = t if acc is None else acc + t
    return acc + ff


def _k_lin_qkv(*refs, D):
    (pts_ref, xyz_ref, w0, b0, w1, b1, wq, bq, wk, bk, wv, bv,
     f_ref, q_ref, kv_ref) = refs
    fz = jnp.dot(pts_ref[0], w0[...], preferred_element_type=F32) + b0[...]
    f_ref[0] = fz
    q_ref[0] = _qkv_from(fz, xyz_ref[0], (w1, b1, wq, bq, wk, bk, wv, bv),
                         kv_ref, D=D)


def _st_lin_qkv(points, xyz, plin, pptb):
    B, N, _ = points.shape
    D = pptb['lin1']['W'].shape[1]
    W = _wpad(D)
    dps = [plin, pptb['lin1'], pptb['ptl']['q'], pptb['ptl']['k'],
           pptb['ptl']['v']]
    wargs = [x for dp in dps for x in (dp['W'], dp['b'].reshape(1, -1))]
    return pl.pallas_call(
        functools.partial(_k_lin_qkv, D=D),
        grid=(B,),
        in_specs=[_bspec(points.shape), _bspec(xyz.shape)]
        + [_full_spec(w.shape) for w in wargs],
        out_specs=[_bspec((B, N, D)), _bspec((B, N, D)), _bspec((B, N, W))],
        out_shape=[jax.ShapeDtypeStruct((B, N, D), F32),
                   jax.ShapeDtypeStruct((B, N, D), F32),
                   jax.ShapeDtypeStruct((B, N, W), F32)],
    )(points, xyz, *wargs)


def _k_td_qkv(*refs, D, Ktd, Nn):
    (fprev_ref, xyz_ref, itd_ref, wtd, btd,
     w1, b1, wq, bq, wk, bk, wv, bv, f_ref, q_ref, kv_ref) = refs
    fn = _td_from(fprev_ref[0], itd_ref, wtd, btd, Ktd, Nn)
    f_ref[0] = fn
    q_ref[0] = _qkv_from(fn, xyz_ref[0], (w1, b1, wq, bq, wk, bk, wv, bv),
                         kv_ref, D=D)


def _st_td_qkv(fprev, xyzq, itd, ptd, pptb, Ktd):
    B, Nprev, Din = fprev.shape
    Nn = xyzq.shape[1]
    D = pptb['lin1']['W'].shape[1]
    W = _wpad(D)
    dps = [ptd['mlp'], pptb['lin1'], pptb['ptl']['q'], pptb['ptl']['k'],
           pptb['ptl']['v']]
    wargs = [x for dp in dps for x in (dp['W'], dp['b'].reshape(1, -1))]
    return pl.pallas_call(
        functools.partial(_k_td_qkv, D=D, Ktd=Ktd, Nn=Nn),
        grid=(B,),
        in_specs=[_bspec(fprev.shape), _bspec(xyzq.shape),
                  _bspec(itd.shape)] + [_full_spec(w.shape) for w in wargs],
        out_specs=[_bspec((B, Nn, D)), _bspec((B, Nn, D)), _bspec((B, Nn, W))],
        out_shape=[jax.ShapeDtypeStruct((B, Nn, D), F32),
                   jax.ShapeDtypeStruct((B, Nn, D), F32),
                   jax.ShapeDtypeStruct((B, Nn, W), F32)],
    )(fprev, xyzq, itd, *wargs)


def _k_tu_qkv(*refs, D):
    (fc_ref, ff_ref, d_ref, itu_ref, xyz_ref, wl1, bl1, wl2, bl2,
     w1, b1, wq, bq, wk, bk, wv, bv, f_ref, q_ref, kv_ref) = refs
    fn = _tu_from(fc_ref[0], ff_ref[0], d_ref, itu_ref, wl1, bl1, wl2, bl2)
    f_ref[0] = fn
    q_ref[0] = _qkv_from(fn, xyz_ref[0], (w1, b1, wq, bq, wk, bk, wv, bv),
                         kv_ref, D=D)


def _st_tu_qkv(fc, ff, d, itu, xyzq, ptu, pptb):
    B, Nf, Df = ff.shape
    D = pptb['lin1']['W'].shape[1]
    W = _wpad(D)
    dps = [ptu['lin1'], ptu['lin2'], pptb['lin1'], pptb['ptl']['q'],
           pptb['ptl']['k'], pptb['ptl']['v']]
    wargs = [x for dp in dps for x in (dp['W'], dp['b'].reshape(1, -1))]
    return pl.pallas_call(
        functools.partial(_k_tu_qkv, D=D),
        grid=(B,),
        in_specs=[_bspec(fc.shape), _bspec(ff.shape), _bspec(d.shape),
                  _bspec(itu.shape), _bspec(xyzq.shape)]
        + [_full_spec(w.shape) for w in wargs],
        out_specs=[_bspec((B, Nf, D)), _bspec((B, Nf, D)), _bspec((B, Nf, W))],
        out_shape=[jax.ShapeDtypeStruct((B, Nf, D), F32),
                   jax.ShapeDtypeStruct((B, Nf, D), F32),
                   jax.ShapeDtypeStruct((B, Nf, W), F32)],
    )(fc, ff, d, itu, xyzq, *wargs)


def _k_td_ptb(*refs, Ktd, K, Nn):
    fprev_ref, xyz_ref, itd_ref, iptb_ref, wtd, btd = refs[:6]
    wrefs = refs[6:24]
    o_ref = refs[24]
    fn = _td_from(fprev_ref[0], itd_ref, wtd, btd, Ktd, Nn)
    o_ref[0] = _ptb_from_refs(fn, xyz_ref[0], iptb_ref, wrefs, K)


def _st_td_ptb(fprev, xyzq, itd, iptb, ptd, pptb, Ktd, K):
    B = fprev.shape[0]
    Nn = xyzq.shape[1]
    D = pptb['lin1']['W'].shape[1]
    wargs = [ptd['mlp']['W'], ptd['mlp']['b'].reshape(1, -1)] + _ptb_wargs(pptb)
    return pl.pallas_call(
        functools.partial(_k_td_ptb, Ktd=Ktd, K=K, Nn=Nn),
        grid=(B,),
        in_specs=[_bspec(fprev.shape), _bspec(xyzq.shape), _bspec(itd.shape),
                  _bspec(iptb.shape)] + [_full_spec(w.shape) for w in wargs],
        out_specs=_bspec((B, Nn, D)),
        out_shape=jax.ShapeDtypeStruct((B, Nn, D), F32),
    )(fprev, xyzq, itd, iptb, *wargs)


def _k_lin_ptb(*refs, K):
    f_ref, xyz_ref, iptb_ref, wl, bl = refs[:5]
    wrefs = refs[5:23]
    o_ref = refs[23]
    fn = jnp.dot(f_ref[0], wl[...], preferred_element_type=F32) + bl[...]
    o_ref[0] = _ptb_from_refs(fn, xyz_ref[0], iptb_ref, wrefs, K)


def _st_lin_ptb(f, xyzq, iptb, plin, pptb, K):
    B, N, _ = f.shape
    D = pptb['lin1']['W'].shape[1]
    wargs = [plin['W'], plin['b'].reshape(1, -1)] + _ptb_wargs(pptb)
    return pl.pallas_call(
        functools.partial(_k_lin_ptb, K=K),
        grid=(B,),
        in_specs=[_bspec(f.shape), _bspec(xyzq.shape), _bspec(iptb.shape)]
        + [_full_spec(w.shape) for w in wargs],
        out_specs=_bspec((B, N, D)),
        out_shape=jax.ShapeDtypeStruct((B, N, D), F32),
    )(f, xyzq, iptb, *wargs)


def _k_tu_ptb(*refs, K):
    fc_ref, ff_ref, d_ref, itu_ref, xyz_ref, iptb_ref, wl1, bl1, wl2, bl2 = (
        refs[:10])
    wrefs = refs[10:28]
    o_ref = refs[28]
    fn = _tu_from(fc_ref[0], ff_ref[0], d_ref, itu_ref, wl1, bl1, wl2, bl2)
    o_ref[0] = _ptb_from_refs(fn, xyz_ref[0], iptb_ref, wrefs, K)


def _st_tu_ptb(fc, ff, d, itu, xyzq, iptb, ptu, pptb, K):
    B, Nf, Df = ff.shape
    D = pptb['lin1']['W'].shape[1]
    wargs = [ptu['lin1']['W'], ptu['lin1']['b'].reshape(1, -1),
             ptu['lin2']['W'], ptu['lin2']['b'].reshape(1, -1)] + \
        _ptb_wargs(pptb)
    return pl.pallas_call(
        functools.partial(_k_tu_ptb, K=K),
        grid=(B,),
        in_specs=[_bspec(fc.shape), _bspec(ff.shape), _bspec(d.shape),
                  _bspec(itu.shape), _bspec(xyzq.shape), _bspec(iptb.shape)]
        + [_full_spec(w.shape) for w in wargs],
        out_specs=_bspec((B, Nf, D)),
        out_shape=jax.ShapeDtypeStruct((B, Nf, D), F32),
    )(fc, ff, d, itu, xyzq, iptb, *wargs)


def _k_attn_g(*refs, K, D, has_mlp):
    if has_mlp:
        (fb_ref, xq_ref, qb_ref, kvg_ref,
         wp1, bp1, wp2, bp2, wa1, ba1, wa2, ba2, w2, b2, wm, bm, o_ref) = refs
    else:
        (fb_ref, xq_ref, qb_ref, kvg_ref,
         wp1, bp1, wp2, bp2, wa1, ba1, wa2, ba2, w2, b2, o_ref) = refs
    qb = qb_ref[0]
    xq = xq_ref[0]
    pairs = []
    for kk in range(K):
        kv = kvg_ref[0, kk]
        pairs.append((kv[:, :D], kv[:, D:2 * D], kv[:, 2 * D:2 * D + 3]))
    attn = _attn_tail(qb, xq, pairs, wp1, bp1, wp2, bp2, wa1, ba1, wa2, ba2)
    fo = fb_ref[0] + (
        jnp.dot(attn, w2[...], preferred_element_type=F32) + b2[...])
    if has_mlp:
        o_ref[0] = jnp.dot(fo, wm[...], preferred_element_type=F32) + bm[...]
    else:
        o_ref[0] = fo


def _pc_attn_g(f, xyz, q, kvg, p, K, pmlp=None):
    B, N, D = f.shape
    R = min(N, 512)
    NB = N // R
    W = _wpad(D)
    args = _ptb_wargs(p)[8:]
    Dout = D
    if pmlp is not None:
        args = args + [pmlp['W'], pmlp['b'].reshape(1, -1)]
        Dout = pmlp['W'].shape[1]
    specs = [
        pl.BlockSpec((1, R, D), lambda b, i: (b, i, 0)),
        pl.BlockSpec((1, R, 3), lambda b, i: (b, i, 0)),
        pl.BlockSpec((1, R, D), lambda b, i: (b, i, 0)),
        pl.BlockSpec((1, K, R, W), lambda b, i: (b, 0, i, 0)),
    ] + [_full_spec(w.shape) for w in args]
    kern = functools.partial(_k_attn_g, K=K, D=D, has_mlp=pmlp is not None)
    return pl.pallas_call(
        kern,
        grid=(B, NB),
        in_specs=specs,
        out_specs=pl.BlockSpec((1, R, Dout), lambda b, i: (b, i, 0)),
        out_shape=jax.ShapeDtypeStruct((B, N, Dout), F32),
    )(f, xyz, q, kvg, *args)


def kernel(points, params):
    p = params
    B = points.shape[0]
    xyz0 = points[:, :, :3]
    xyz1 = xyz0[:, :512]
    xyz2 = xyz0[:, :128]
    xyz3 = xyz0[:, :32]
    xyz4 = xyz0[:, :8]

    (ig0, ig1, ig2, i3, i4, itd1, itd2, itd3, itd4,
     d6, i6, d7, i7, d8, i8, d9, i9) = _knn_all(xyz0)

    f, q0, kvx0 = _st_lin_qkv(points, xyz0, p['lin0'], p['ptb0'])
    kvg0 = _gather_rows(kvx0.reshape(B * 2048, 128), ig0, 128)
    f0 = _pc_attn_g(f, xyz0, q0, kvg0, p['ptb0'], 16)

    f1a, q1, kvx1 = _st_td_qkv(f0, xyz1, itd1, p['tdb1'], p['ptb1'], 16)
    kvg1 = _gather_rows(kvx1.reshape(B * 512, 256), ig1, 256)
    f1 = _pc_attn_g(f1a, xyz1, q1, kvg1, p['ptb1'], 16)

    f2a, q2, kvx2 = _st_td_qkv(f1, xyz2, itd2, p['tdb2'], p['ptb2'], 8)
    kvg2 = _gather_rows(kvx2.reshape(B * 128, 384), ig2, 384)
    f2 = _pc_attn_g(f2a, xyz2, q2, kvg2, p['ptb2'], 8)

    f3 = _st_td_ptb(f2, xyz3, itd3, i3, p['tdb3'], p['ptb3'], 4, 4)
    f4 = _st_td_ptb(f3, xyz4, itd4, i4, p['tdb4'], p['ptb4'], 2, 2)
    f4 = _st_lin_ptb(f4, xyz4, i4, p['lin2'], p['ptb5'], 2)
    fu = _st_tu_ptb(f4, f3, d6, i6, xyz3, i3, p['tub6'], p['ptb6'], 4)

    fu7, q7, kvx7 = _st_tu_qkv(fu, f2, d7, i7, xyz2, p['tub7'], p['ptb7'])
    kvg7 = _gather_rows(kvx7.reshape(B * 128, 384), ig2, 384)
    fu = _pc_attn_g(fu7, xyz2, q7, kvg7, p['ptb7'], 8)

    fu8, q8, kvx8 = _st_tu_qkv(fu, f1, d8, i8, xyz1, p['tub8'], p['ptb8'])
    kvg8 = _gather_rows(kvx8.reshape(B * 512, 256), ig1, 256)
    fu = _pc_attn_g(fu8, xyz1, q8, kvg8, p['ptb8'], 16)

    fu9, q9, kvx9 = _st_tu_qkv(fu, f0, d9, i9, xyz0, p['tub9'], p['ptb9'])
    kvg9 = _gather_rows(kvx9.reshape(B * 2048, 128), ig0, 128)
    out = _pc_attn_g(fu9, xyz0, q9, kvg9, p['ptb9'], 16, pmlp=p['mlp'])
    return xyz0, out


# speedup vs baseline: 1.0002x; 1.0002x over previous
import functools

import jax
import jax.numpy as jnp
from jax.experimental import pallas as pl
from jax.experimental.pallas import tpu as pltpu
from jax.experimental.pallas import tpu_sc as plsc

F32 = jnp.float32

_SC_NC = 2
_SC_NS = 16
_SC_NW = _SC_NC * _SC_NS
_SC_L = 128


def _wpad(D):
    return ((2 * D + 3 + 127) // 128) * 128


def _full_spec(shape):
    return pl.BlockSpec(shape, lambda *a, n=len(shape): (0,) * n)


def _bspec(shape):
    return pl.BlockSpec((1,) + tuple(shape[1:]),
                        lambda *a, n=len(shape): (a[0],) + (0,) * (n - 1))


def _ptb_wargs(p):
    ptl = p['ptl']
    dps = [p['lin1'], ptl['q'], ptl['k'], ptl['v'],
           ptl['pos']['l1'], ptl['pos']['l2'],
           ptl['attn']['l1'], ptl['attn']['l2'], p['lin2']]
    return [x for dp in dps for x in (dp['W'], dp['b'].reshape(1, -1))]


def _sc_gather(table, idx_flat, C):
    M = idx_flat.shape[0]
    nck = M // (_SC_NW * _SC_L)
    idx3 = idx_flat.reshape(_SC_NW, nck, _SC_L)
    mesh = plsc.VectorSubcoreMesh(core_axis_name="c", subcore_axis_name="s")

    def body(idx_hbm, tab_hbm, out_hbm, idx_v, rows_v, sem):
        wid = jax.lax.axis_index("s") * _SC_NC + jax.lax.axis_index("c")

        def chunk(c, carry):
            pltpu.sync_copy(idx_hbm.at[wid, c], idx_v)
            pltpu.async_copy(tab_hbm.at[idx_v], rows_v, sem).wait()
            pltpu.sync_copy(
                rows_v, out_hbm.at[pl.ds((wid * nck + c) * _SC_L, _SC_L)])
            return carry

        jax.lax.fori_loop(0, nck, chunk, 0)

    fn = pl.kernel(
        body,
        mesh=mesh,
        out_type=jax.ShapeDtypeStruct((M, C), F32),
        scratch_types=[
            pltpu.VMEM((_SC_L,), jnp.int32),
            pltpu.VMEM((_SC_L, C), F32),
            pltpu.SemaphoreType.DMA,
        ],
    )
    return fn(idx3, table)


def _gather_rows(table2d, idxg, C):
    B, K, Nq = idxg.shape
    M = B * K * Nq
    Mp = ((M + 4095) // 4096) * 4096
    flat = idxg.reshape(M)
    if Mp != M:
        flat = jnp.pad(flat, (0, Mp - M))
    g = _sc_gather(table2d, flat, C)
    return g[:M].reshape(B, K, Nq, C)


_KNN_CFGS = (
    (2048, 2048, 16, False, False, True),
    (512, 512, 16, False, False, True),
    (128, 128, 8, False, False, True),
    (32, 32, 4, False, True, False),
    (8, 8, 2, False, True, False),
    (512, 2048, 16, False, True, False),
    (128, 512, 8, False, True, False),
    (32, 128, 4, False, True, False),
    (8, 32, 2, False, True, False),
    (32, 8, 3, True, True, False),
    (128, 32, 3, True, True, False),
    (512, 128, 3, True, True, False),
    (2048, 512, 3, True, True, False),
)


def _knn_all_kernel(xyz_ref, xyzT_ref, *out_refs):
    b = pl.program_id(0)
    oi = 0
    for (Nq, Nk, K, wd, wl, wg) in _KNN_CFGS:
        dref = iref = gref = None
        if wd:
            dref = out_refs[oi]
            oi += 1
        if wl:
            iref = out_refs[oi]
            oi += 1
        if wg:
            gref = out_refs[oi]
            oi += 1
        R = min(Nq, 512)
        for blk in range(Nq // R):
            r0 = blk * R
            qx = xyz_ref[0, r0:r0 + R, 0][:, None]
            qy = xyz_ref[0, r0:r0 + R, 1][:, None]
            qz = xyz_ref[0, r0:r0 + R, 2][:, None]
            kx = xyzT_ref[0, 0, :Nk][None, :]
            ky = xyzT_ref[0, 1, :Nk][None, :]
            kz = xyzT_ref[0, 2, :Nk][None, :]
            dxv = qx - kx
            dyv = qy - ky
            dzv = qz - kz
            cur = dxv * dxv + dyv * dyv + dzv * dzv
            iota = jax.lax.broadcasted_iota(jnp.int32, (R, Nk), 1)
            for kk in range(K):
                m = jnp.min(cur, axis=1)
                am = jnp.min(jnp.where(cur == m[:, None], iota, Nk), axis=1)
                if wd:
                    dref[0, kk, r0:r0 + R] = m
                if wl:
                    iref[0, kk, r0:r0 + R] = am
                if wg:
                    gref[0, kk, r0:r0 + R] = am + b * Nk
                if kk < K - 1:
                    cur = jnp.where(iota == am[:, None],
                                    jnp.float32(jnp.inf), cur)


def _knn_all(xyz0):
    B = xyz0.shape[0]
    xyzT = jnp.transpose(xyz0, (0, 2, 1))
    out_shape = []
    out_specs = []
    for (Nq, Nk, K, wd, wl, wg) in _KNN_CFGS:
        if wd:
            out_shape.append(jax.ShapeDtypeStruct((B, K, Nq), F32))
            out_specs.append(_bspec((B, K, Nq)))
        if wl:
            out_shape.append(jax.ShapeDtypeStruct((B, K, Nq), jnp.int32))
            out_specs.append(_bspec((B, K, Nq)))
        if wg:
            out_shape.append(jax.ShapeDtypeStruct((B, K, Nq), jnp.int32))
            out_specs.append(_bspec((B, K, Nq)))
    return pl.pallas_call(
        _knn_all_kernel,
        grid=(B,),
        compiler_params=pltpu.CompilerParams(
            dimension_semantics=("parallel",)),
        in_specs=[_bspec(xyz0.shape), _bspec(xyzT.shape)],
        out_specs=out_specs,
        out_shape=out_shape,
    )(xyz0, xyzT)


def _attn_tail(qb, xq, pairs, wp1, bp1, wp2, bp2, wa1, ba1, wa2, ba2):
    a_list = []
    vp_list = []
    for kg, vg, nx in pairs:
        pd = xq - nx
        h = jnp.maximum(jnp.dot(pd, wp1[...], preferred_element_type=F32)
                        + bp1[...], 0.0)
        pos = jnp.dot(h, wp2[...], preferred_element_type=F32) + bp2[...]
        ain = qb - kg + pos
        h2 = jnp.maximum(jnp.dot(ain, wa1[...], preferred_element_type=F32)
                         + ba1[...], 0.0)
        a = jnp.dot(h2, wa2[...], preferred_element_type=F32) + ba2[...]
        a_list.append(a)
        vp_list.append(vg + pos)
    m = a_list[0]
    for a in a_list[1:]:
        m = jnp.maximum(m, a)
    es = [jnp.exp(a - m) for a in a_list]
    s = es[0]
    for e in es[1:]:
        s = s + e
    num = es[0] * vp_list[0]
    for kk in range(1, len(es)):
        num = num + es[kk] * vp_list[kk]
    return num / s


def _qkv_from(y_in, xyz, wrefs, kv_ref, *, D):
    (w1, b1, wq, bq, wk, bk, wv, bv) = wrefs
    y = jnp.dot(y_in, w1[...], preferred_element_type=F32) + b1[...]
    q = jnp.dot(y, wq[...], preferred_element_type=F32) + bq[...]
    kv_ref[0, :, :D] = jnp.dot(y, wk[...], preferred_element_type=F32) + bk[...]
    kv_ref[0, :, D:2 * D] = (
        jnp.dot(y, wv[...], preferred_element_type=F32) + bv[...])
    kv_ref[0, :, 2 * D:2 * D + 3] = xyz
    return q


def _ptb_from_refs(f, xyz, idx_ref, wrefs, K):
    (w1, b1, wq, bq, wk, bk, wv, bv,
     wp1, bp1, wp2, bp2, wa1, ba1, wa2, ba2, w2, b2) = wrefs
    N = f.shape[0]
    y = jnp.dot(f, w1[...], preferred_element_type=F32) + b1[...]
    q = jnp.dot(y, wq[...], preferred_element_type=F32) + bq[...]
    k = jnp.dot(y, wk[...], preferred_element_type=F32) + bk[...]
    v = jnp.dot(y, wv[...], preferred_element_type=F32) + bv[...]
    iota = jax.lax.broadcasted_iota(jnp.int32, (N, N), 1)
    pairs = []
    for kk in range(K):
        oh = (idx_ref[0, kk, :][:, None] == iota).astype(F32)
        kg = jnp.dot(oh, k, preferred_element_type=F32)
        vg = jnp.dot(oh, v, preferred_element_type=F32)
        nx = jnp.dot(oh, xyz, preferred_element_type=F32)
        pairs.append((kg, vg, nx))
    attn = _attn_tail(q, xyz, pairs, wp1, bp1, wp2, bp2, wa1, ba1, wa2, ba2)
    return f + jnp.dot(attn, w2[...], preferred_element_type=F32) + b2[...]


def _td_from(fprev, itd_ref, wtd, btd, Ktd, Nn):
    Nprev = fprev.shape[0]
    iota = jax.lax.broadcasted_iota(jnp.int32, (Nn, Nprev), 1)
    acc = None
    for kk in range(Ktd):
        oh = (itd_ref[0, kk, :][:, None] == iota).astype(F32)
        g = jnp.dot(oh, fprev, preferred_element_type=F32)
        h = jnp.maximum(
            jnp.dot(g, wtd[...], preferred_element_type=F32) + btd[...], 0.0)
        acc = h if acc is None else jnp.maximum(acc, h)
    return acc


def _tu_from(fc_in, ff_in, d_ref, itu_ref, wl1, bl1, wl2, bl2):
    Nc = fc_in.shape[0]
    Nf = ff_in.shape[0]
    fc = jnp.dot(fc_in, wl1[...], preferred_element_type=F32) + bl1[...]
    ff = jnp.dot(ff_in, wl2[...], preferred_element_type=F32) + bl2[...]
    ws = [1.0 / (d_ref[0, kk, :] + 1e-8) for kk in range(3)]
    s = ws[0] + ws[1] + ws[2]
    iota = jax.lax.broadcasted_iota(jnp.int32, (Nf, Nc), 1)
    acc = None
    for kk in range(3):
        oh = (itu_ref[0, kk, :][:, None] == iota).astype(F32)
        g = jnp.dot(oh, fc, preferred_element_type=F32)
        t = (ws[kk] / s)[:, None] * g
        acc = t if acc is None else acc + t
    return acc + ff


def _k_lin_qkv(*refs, D):
    (pts_ref, xyz_ref, w0, b0, w1, b1, wq, bq, wk, bk, wv, bv,
     f_ref, q_ref, kv_ref) = refs
    fz = jnp.dot(pts_ref[0], w0[...], preferred_element_type=F32) + b0[...]
    f_ref[0] = fz
    q_ref[0] = _qkv_from(fz, xyz_ref[0], (w1, b1, wq, bq, wk, bk, wv, bv),
                         kv_ref, D=D)


def _st_lin_qkv(points, xyz, plin, pptb):
    B, N, _ = points.shape
    D = pptb['lin1']['W'].shape[1]
    W = _wpad(D)
    dps = [plin, pptb['lin1'], pptb['ptl']['q'], pptb['ptl']['k'],
           pptb['ptl']['v']]
    wargs = [x for dp in dps for x in (dp['W'], dp['b'].reshape(1, -1))]
    return pl.pallas_call(
        functools.partial(_k_lin_qkv, D=D),
        grid=(B,),
        compiler_params=pltpu.CompilerParams(
            dimension_semantics=("parallel",)),
        in_specs=[_bspec(points.shape), _bspec(xyz.shape)]
        + [_full_spec(w.shape) for w in wargs],
        out_specs=[_bspec((B, N, D)), _bspec((B, N, D)), _bspec((B, N, W))],
        out_shape=[jax.ShapeDtypeStruct((B, N, D), F32),
                   jax.ShapeDtypeStruct((B, N, D), F32),
                   jax.ShapeDtypeStruct((B, N, W), F32)],
    )(points, xyz, *wargs)


def _k_td_qkv(*refs, D, Ktd, Nn):
    (fprev_ref, xyz_ref, itd_ref, wtd, btd,
     w1, b1, wq, bq, wk, bk, wv, bv, f_ref, q_ref, kv_ref) = refs
    fn = _td_from(fprev_ref[0], itd_ref, wtd, btd, Ktd, Nn)
    f_ref[0] = fn
    q_ref[0] = _qkv_from(fn, xyz_ref[0], (w1, b1, wq, bq, wk, bk, wv, bv),
                         kv_ref, D=D)


def _st_td_qkv(fprev, xyzq, itd, ptd, pptb, Ktd):
    B, Nprev, Din = fprev.shape
    Nn = xyzq.shape[1]
    D = pptb['lin1']['W'].shape[1]
    W = _wpad(D)
    dps = [ptd['mlp'], pptb['lin1'], pptb['ptl']['q'], pptb['ptl']['k'],
           pptb['ptl']['v']]
    wargs = [x for dp in dps for x in (dp['W'], dp['b'].reshape(1, -1))]
    return pl.pallas_call(
        functools.partial(_k_td_qkv, D=D, Ktd=Ktd, Nn=Nn),
        grid=(B,),
        compiler_params=pltpu.CompilerParams(
            dimension_semantics=("parallel",)),
        in_specs=[_bspec(fprev.shape), _bspec(xyzq.shape),
                  _bspec(itd.shape)] + [_full_spec(w.shape) for w in wargs],
        out_specs=[_bspec((B, Nn, D)), _bspec((B, Nn, D)), _bspec((B, Nn, W))],
        out_shape=[jax.ShapeDtypeStruct((B, Nn, D), F32),
                   jax.ShapeDtypeStruct((B, Nn, D), F32),
                   jax.ShapeDtypeStruct((B, Nn, W), F32)],
    )(fprev, xyzq, itd, *wargs)


def _k_tu_qkv(*refs, D):
    (fc_ref, ff_ref, d_ref, itu_ref, xyz_ref, wl1, bl1, wl2, bl2,
     w1, b1, wq, bq, wk, bk, wv, bv, f_ref, q_ref, kv_ref) = refs
    fn = _tu_from(fc_ref[0], ff_ref[0], d_ref, itu_ref, wl1, bl1, wl2, bl2)
    f_ref[0] = fn
    q_ref[0] = _qkv_from(fn, xyz_ref[0], (w1, b1, wq, bq, wk, bk, wv, bv),
                         kv_ref, D=D)


def _st_tu_qkv(fc, ff, d, itu, xyzq, ptu, pptb):
    B, Nf, Df = ff.shape
    D = pptb['lin1']['W'].shape[1]
    W = _wpad(D)
    dps = [ptu['lin1'], ptu['lin2'], pptb['lin1'], pptb['ptl']['q'],
           pptb['ptl']['k'], pptb['ptl']['v']]
    wargs = [x for dp in dps for x in (dp['W'], dp['b'].reshape(1, -1))]
    return pl.pallas_call(
        functools.partial(_k_tu_qkv, D=D),
        grid=(B,),
        compiler_params=pltpu.CompilerParams(
            dimension_semantics=("parallel",)),
        in_specs=[_bspec(fc.shape), _bspec(ff.shape), _bspec(d.shape),
                  _bspec(itu.shape), _bspec(xyzq.shape)]
        + [_full_spec(w.shape) for w in wargs],
        out_specs=[_bspec((B, Nf, D)), _bspec((B, Nf, D)), _bspec((B, Nf, W))],
        out_shape=[jax.ShapeDtypeStruct((B, Nf, D), F32),
                   jax.ShapeDtypeStruct((B, Nf, D), F32),
                   jax.ShapeDtypeStruct((B, Nf, W), F32)],
    )(fc, ff, d, itu, xyzq, *wargs)


def _k_td_ptb(*refs, Ktd, K, Nn):
    fprev_ref, xyz_ref, itd_ref, iptb_ref, wtd, btd = refs[:6]
    wrefs = refs[6:24]
    o_ref = refs[24]
    fn = _td_from(fprev_ref[0], itd_ref, wtd, btd, Ktd, Nn)
    o_ref[0] = _ptb_from_refs(fn, xyz_ref[0], iptb_ref, wrefs, K)


def _st_td_ptb(fprev, xyzq, itd, iptb, ptd, pptb, Ktd, K):
    B = fprev.shape[0]
    Nn = xyzq.shape[1]
    D = pptb['lin1']['W'].shape[1]
    wargs = [ptd['mlp']['W'], ptd['mlp']['b'].reshape(1, -1)] + _ptb_wargs(pptb)
    return pl.pallas_call(
        functools.partial(_k_td_ptb, Ktd=Ktd, K=K, Nn=Nn),
        grid=(B,),
        compiler_params=pltpu.CompilerParams(
            dimension_semantics=("parallel",)),
        in_specs=[_bspec(fprev.shape), _bspec(xyzq.shape), _bspec(itd.shape),
                  _bspec(iptb.shape)] + [_full_spec(w.shape) for w in wargs],
        out_specs=_bspec((B, Nn, D)),
        out_shape=jax.ShapeDtypeStruct((B, Nn, D), F32),
    )(fprev, xyzq, itd, iptb, *wargs)


def _k_lin_ptb(*refs, K):
    f_ref, xyz_ref, iptb_ref, wl, bl = refs[:5]
    wrefs = refs[5:23]
    o_ref = refs[23]
    fn = jnp.dot(f_ref[0], wl[...], preferred_element_type=F32) + bl[...]
    o_ref[0] = _ptb_from_refs(fn, xyz_ref[0], iptb_ref, wrefs, K)


def _st_lin_ptb(f, xyzq, iptb, plin, pptb, K):
    B, N, _ = f.shape
    D = pptb['lin1']['W'].shape[1]
    wargs = [plin['W'], plin['b'].reshape(1, -1)] + _ptb_wargs(pptb)
    return pl.pallas_call(
        functools.partial(_k_lin_ptb, K=K),
        grid=(B,),
        compiler_params=pltpu.CompilerParams(
            dimension_semantics=("parallel",)),
        in_specs=[_bspec(f.shape), _bspec(xyzq.shape), _bspec(iptb.shape)]
        + [_full_spec(w.shape) for w in wargs],
        out_specs=_bspec((B, N, D)),
        out_shape=jax.ShapeDtypeStruct((B, N, D), F32),
    )(f, xyzq, iptb, *wargs)


def _k_tu_ptb(*refs, K):
    fc_ref, ff_ref, d_ref, itu_ref, xyz_ref, iptb_ref, wl1, bl1, wl2, bl2 = (
        refs[:10])
    wrefs = refs[10:28]
    o_ref = refs[28]
    fn = _tu_from(fc_ref[0], ff_ref[0], d_ref, itu_ref, wl1, bl1, wl2, bl2)
    o_ref[0] = _ptb_from_refs(fn, xyz_ref[0], iptb_ref, wrefs, K)


def _st_tu_ptb(fc, ff, d, itu, xyzq, iptb, ptu, pptb, K):
    B, Nf, Df = ff.shape
    D = pptb['lin1']['W'].shape[1]
    wargs = [ptu['lin1']['W'], ptu['lin1']['b'].reshape(1, -1),
             ptu['lin2']['W'], ptu['lin2']['b'].reshape(1, -1)] + \
        _ptb_wargs(pptb)
    return pl.pallas_call(
        functools.partial(_k_tu_ptb, K=K),
        grid=(B,),
        compiler_params=pltpu.CompilerParams(
            dimension_semantics=("parallel",)),
        in_specs=[_bspec(fc.shape), _bspec(ff.shape), _bspec(d.shape),
                  _bspec(itu.shape), _bspec(xyzq.shape), _bspec(iptb.shape)]
        + [_full_spec(w.shape) for w in wargs],
        out_specs=_bspec((B, Nf, D)),
        out_shape=jax.ShapeDtypeStruct((B, Nf, D), F32),
    )(fc, ff, d, itu, xyzq, iptb, *wargs)


def _k_attn_g(*refs, K, D, has_mlp):
    if has_mlp:
        (fb_ref, xq_ref, qb_ref, kvg_ref,
         wp1, bp1, wp2, bp2, wa1, ba1, wa2, ba2, w2, b2, wm, bm, o_ref) = refs
    else:
        (fb_ref, xq_ref, qb_ref, kvg_ref,
         wp1, bp1, wp2, bp2, wa1, ba1, wa2, ba2, w2, b2, o_ref) = refs
    qb = qb_ref[0]
    xq = xq_ref[0]
    pairs = []
    for kk in range(K):
        kv = kvg_ref[0, kk]
        pairs.append((kv[:, :D], kv[:, D:2 * D], kv[:, 2 * D:2 * D + 3]))
    attn = _attn_tail(qb, xq, pairs, wp1, bp1, wp2, bp2, wa1, ba1, wa2, ba2)
    fo = fb_ref[0] + (
        jnp.dot(attn, w2[...], preferred_element_type=F32) + b2[...])
    if has_mlp:
        o_ref[0] = jnp.dot(fo, wm[...], preferred_element_type=F32) + bm[...]
    else:
        o_ref[0] = fo


def _pc_attn_g(f, xyz, q, kvg, p, K, pmlp=None):
    B, N, D = f.shape
    R = min(N, 512)
    NB = N // R
    W = _wpad(D)
    args = _ptb_wargs(p)[8:]
    Dout = D
    if pmlp is not None:
        args = args + [pmlp['W'], pmlp['b'].reshape(1, -1)]
        Dout = pmlp['W'].shape[1]
    specs = [
        pl.BlockSpec((1, R, D), lambda b, i: (b, i, 0)),
        pl.BlockSpec((1, R, 3), lambda b, i: (b, i, 0)),
        pl.BlockSpec((1, R, D), lambda b, i: (b, i, 0)),
        pl.BlockSpec((1, K, R, W), lambda b, i: (b, 0, i, 0)),
    ] + [_full_spec(w.shape) for w in args]
    kern = functools.partial(_k_attn_g, K=K, D=D, has_mlp=pmlp is not None)
    return pl.pallas_call(
        kern,
        grid=(B, NB),
        compiler_params=pltpu.CompilerParams(
            dimension_semantics=("parallel", "parallel")),
        in_specs=specs,
        out_specs=pl.BlockSpec((1, R, Dout), lambda b, i: (b, i, 0)),
        out_shape=jax.ShapeDtypeStruct((B, N, Dout), F32),
    )(f, xyz, q, kvg, *args)


def kernel(points, params):
    p = params
    B = points.shape[0]
    xyz0 = points[:, :, :3]
    xyz1 = xyz0[:, :512]
    xyz2 = xyz0[:, :128]
    xyz3 = xyz0[:, :32]
    xyz4 = xyz0[:, :8]

    (ig0, ig1, ig2, i3, i4, itd1, itd2, itd3, itd4,
     d6, i6, d7, i7, d8, i8, d9, i9) = _knn_all(xyz0)

    f, q0, kvx0 = _st_lin_qkv(points, xyz0, p['lin0'], p['ptb0'])
    kvg0 = _gather_rows(kvx0.reshape(B * 2048, 128), ig0, 128)
    f0 = _pc_attn_g(f, xyz0, q0, kvg0, p['ptb0'], 16)

    f1a, q1, kvx1 = _st_td_qkv(f0, xyz1, itd1, p['tdb1'], p['ptb1'], 16)
    kvg1 = _gather_rows(kvx1.reshape(B * 512, 256), ig1, 256)
    f1 = _pc_attn_g(f1a, xyz1, q1, kvg1, p['ptb1'], 16)

    f2a, q2, kvx2 = _st_td_qkv(f1, xyz2, itd2, p['tdb2'], p['ptb2'], 8)
    kvg2 = _gather_rows(kvx2.reshape(B * 128, 384), ig2, 384)
    f2 = _pc_attn_g(f2a, xyz2, q2, kvg2, p['ptb2'], 8)

    f3 = _st_td_ptb(f2, xyz3, itd3, i3, p['tdb3'], p['ptb3'], 4, 4)
    f4 = _st_td_ptb(f3, xyz4, itd4, i4, p['tdb4'], p['ptb4'], 2, 2)
    f4 = _st_lin_ptb(f4, xyz4, i4, p['lin2'], p['ptb5'], 2)
    fu = _st_tu_ptb(f4, f3, d6, i6, xyz3, i3, p['tub6'], p['ptb6'], 4)

    fu7, q7, kvx7 = _st_tu_qkv(fu, f2, d7, i7, xyz2, p['tub7'], p['ptb7'])
    kvg7 = _gather_rows(kvx7.reshape(B * 128, 384), ig2, 384)
    fu = _pc_attn_g(fu7, xyz2, q7, kvg7, p['ptb7'], 8)

    fu8, q8, kvx8 = _st_tu_qkv(fu, f1, d8, i8, xyz1, p['tub8'], p['ptb8'])
    kvg8 = _gather_rows(kvx8.reshape(B * 512, 256), ig1, 256)
    fu = _pc_attn_g(fu8, xyz1, q8, kvg8, p['ptb8'], 16)

    fu9, q9, kvx9 = _st_tu_qkv(fu, f0, d9, i9, xyz0, p['tub9'], p['ptb9'])
    kvg9 = _gather_rows(kvx9.reshape(B * 2048, 128), ig0, 128)
    out = _pc_attn_g(fu9, xyz0, q9, kvg9, p['ptb9'], 16, pmlp=p['mlp'])
    return xyz0, out
